# Initial kernel scaffold; baseline (speedup 1.0000x reference)
import functools

import jax
import jax.numpy as jnp
from jax import lax
from jax.experimental import pallas as pl
from jax.experimental.pallas import tpu as pltpu
from jax.experimental.pallas import tpu_sc as plsc

NC = 2
NS = 16
NW = NC * NS
CH = 128
CW = 8


def _dot_t(a, w):
    return lax.dot_general(a, w, (((1,), (1,)), ((), ())),
                           preferred_element_type=jnp.float32)


def _transform(xp, w_l, w_r, b, br):
    npad, d = xp.shape
    grid = npad // br

    def body(x_ref, wl_ref, wr_ref, b_ref, z_ref, r_ref):
        xb = x_ref[...]
        z_ref[...] = _dot_t(xb, wl_ref[...])
        r_ref[...] = _dot_t(xb, wr_ref[...]) + b_ref[...]

    return pl.pallas_call(
        body,
        grid=(grid,),
        in_specs=[
            pl.BlockSpec((br, d), lambda i: (i, 0)),
            pl.BlockSpec((d, d), lambda i: (0, 0)),
            pl.BlockSpec((d, d), lambda i: (0, 0)),
            pl.BlockSpec((1, d), lambda i: (0, 0)),
        ],
        out_specs=[
            pl.BlockSpec((br, d), lambda i: (i, 0)),
            pl.BlockSpec((br, d), lambda i: (i, 0)),
        ],
        out_shape=[
            jax.ShapeDtypeStruct((npad, d), jnp.float32),
            jax.ShapeDtypeStruct((npad, d), jnp.float32),
        ],
    )(xp, w_l, w_r, b.reshape(1, d))


def _combine_transform(p, c, r1, w_l, w_r, b, br):
    npad, d = r1.shape
    grid = npad // br

    def body(p_ref, c_ref, r1_ref, wl_ref, wr_ref, b_ref, z_ref, r_ref):
        agg = p_ref[0] + p_ref[1]
        cnt = jnp.maximum(c_ref[0, :, 0:1] + c_ref[1, :, 0:1], 1.0)
        h = jnp.maximum(agg / cnt + r1_ref[...], 0.0)
        z_ref[...] = _dot_t(h, wl_ref[...])
        r_ref[...] = _dot_t(h, wr_ref[...]) + b_ref[...]

    return pl.pallas_call(
        body,
        grid=(grid,),
        in_specs=[
            pl.BlockSpec((2, br, d), lambda i: (0, i, 0)),
            pl.BlockSpec((2, br, CW), lambda i: (0, i, 0)),
            pl.BlockSpec((br, d), lambda i: (i, 0)),
            pl.BlockSpec((d, d), lambda i: (0, 0)),
            pl.BlockSpec((d, d), lambda i: (0, 0)),
            pl.BlockSpec((1, d), lambda i: (0, 0)),
        ],
        out_specs=[
            pl.BlockSpec((br, d), lambda i: (i, 0)),
            pl.BlockSpec((br, d), lambda i: (i, 0)),
        ],
        out_shape=[
            jax.ShapeDtypeStruct((npad, d), jnp.float32),
            jax.ShapeDtypeStruct((npad, d), jnp.float32),
        ],
    )(p, c, r1, w_l, w_r, b.reshape(1, d))


def _final(p, c, r2, br):
    npad, d = r2.shape
    grid = npad // br

    def body(p_ref, c_ref, r2_ref, o_ref):
        agg = p_ref[0] + p_ref[1]
        cnt = jnp.maximum(c_ref[0, :, 0:1] + c_ref[1, :, 0:1], 1.0)
        o_ref[...] = agg / cnt + r2_ref[...]

    return pl.pallas_call(
        body,
        grid=(grid,),
        in_specs=[
            pl.BlockSpec((2, br, d), lambda i: (0, i, 0)),
            pl.BlockSpec((2, br, CW), lambda i: (0, i, 0)),
            pl.BlockSpec((br, d), lambda i: (i, 0)),
        ],
        out_specs=pl.BlockSpec((br, d), lambda i: (i, 0)),
        out_shape=jax.ShapeDtypeStruct((npad, d), jnp.float32),
    )(p, c, r2)


def _sc_scatter(z, src, dst, zeros_c, zcnt_c, ones_c, nch, compute_counts):
    npad, d = z.shape
    rows_pt = npad // NS
    mesh = plsc.VectorSubcoreMesh(core_axis_name="c", subcore_axis_name="s")

    out_type = [jax.ShapeDtypeStruct((NC, npad, d), jnp.float32)]
    if compute_counts:
        out_type.append(jax.ShapeDtypeStruct((NC, npad, CW), jnp.float32))

    scratch = dict(
        agg_sh=pltpu.VMEM_SHARED((npad, d), jnp.float32),
        src_v=pltpu.VMEM((CH,), jnp.int32),
        dst_v=pltpu.VMEM((CH,), jnp.int32),
        rows_v=pltpu.VMEM((CH, d), jnp.float32),
        sem=pltpu.SemaphoreType.DMA,
    )
    if compute_counts:
        scratch.update(
            cnt_sh=pltpu.VMEM_SHARED((npad, CW), jnp.float32),
            ones_v=pltpu.VMEM((CH, CW), jnp.float32),
        )

    @functools.partial(pl.kernel, out_type=out_type, mesh=mesh,
                       scratch_types=scratch)
    def k(z_hbm, src_hbm, dst_hbm, zeros_hbm, zcnt_hbm, ones_hbm,
          p_hbm, *rest, agg_sh, src_v, dst_v, rows_v, sem,
          cnt_sh=None, ones_v=None):
        cid = lax.axis_index("c")
        sid = lax.axis_index("s")
        wid = sid * NC + cid
        r0 = sid * rows_pt

        pltpu.sync_copy(zeros_hbm, agg_sh.at[pl.ds(r0, rows_pt)])
        if compute_counts:
            pltpu.sync_copy(zcnt_hbm, cnt_sh.at[pl.ds(r0, rows_pt)])
            pltpu.sync_copy(ones_hbm, ones_v)
        plsc.subcore_barrier()

        @pl.loop(0, nch)
        def _(ch):
            base = (wid * nch + ch) * CH
            pltpu.sync_copy(src_hbm.at[pl.ds(base, CH)], src_v)
            pltpu.sync_copy(dst_hbm.at[pl.ds(base, CH)], dst_v)
            pltpu.async_copy(z_hbm.at[src_v], rows_v, sem).wait()
            pltpu.sync_copy(rows_v, agg_sh.at[dst_v], add=True)
            if compute_counts:
                pltpu.sync_copy(ones_v, cnt_sh.at[dst_v], add=True)

        plsc.subcore_barrier()
        pltpu.sync_copy(agg_sh.at[pl.ds(r0, rows_pt)],
                        p_hbm.at[cid, pl.ds(r0, rows_pt)])
        if compute_counts:
            (c_hbm,) = rest
            pltpu.sync_copy(cnt_sh.at[pl.ds(r0, rows_pt)],
                            c_hbm.at[cid, pl.ds(r0, rows_pt)])

    return k(z, src, dst, zeros_c, zcnt_c, ones_c)


def kernel(x, edge_index, W1_l, W1_r, b1, W2_l, W2_r, b2):
    n, d = x.shape
    e = edge_index.shape[1]

    npad = ((n + 1 + NS * 8 - 1) // (NS * 8)) * (NS * 8)
    rows_pt = npad // NS
    br = npad // 8

    per_w = ((e + NW * CH - 1) // (NW * CH)) * CH
    nch = per_w // CH
    e_pad = per_w * NW

    src = jnp.pad(edge_index[0].astype(jnp.int32), (0, e_pad - e))
    dst = jnp.pad(edge_index[1].astype(jnp.int32), (0, e_pad - e),
                  constant_values=n)
    xp = jnp.pad(x, ((0, npad - n), (0, 0)))

    zeros_c = jnp.zeros((rows_pt, d), jnp.float32)
    zcnt_c = jnp.zeros((rows_pt, CW), jnp.float32)
    ones_c = jnp.ones((CH, CW), jnp.float32)

    z1, r1 = _transform(xp, W1_l, W1_r, b1, br)
    p1, c = _sc_scatter(z1, src, dst, zeros_c, zcnt_c, ones_c, nch, True)
    z2, r2 = _combine_transform(p1, c, r1, W2_l, W2_r, b2, br)
    (p2,) = _sc_scatter(z2, src, dst, zeros_c, zcnt_c, ones_c, nch, False)
    out = _final(p2, c, r2, br)
    return out[:n]


# speedup vs baseline: 4.4067x; 4.4067x over previous
import functools

import jax
import jax.numpy as jnp
from jax import lax
from jax.experimental import pallas as pl
from jax.experimental.pallas import tpu as pltpu
from jax.experimental.pallas import tpu_sc as plsc

NC = 2
NS = 16
NW = NC * NS
CH = 128


def _dot_t(a, w):
    return lax.dot_general(a, w, (((1,), (1,)), ((), ())),
                           preferred_element_type=jnp.float32)


def _transform(xp, w_l, w_r, b, br):
    npad, d = xp.shape
    grid = npad // br

    def body(x_ref, wl_ref, wr_ref, b_ref, z_ref, r_ref):
        xb = x_ref[...]
        z_ref[...] = _dot_t(xb, wl_ref[...])
        r_ref[...] = _dot_t(xb, wr_ref[...]) + b_ref[...]

    return pl.pallas_call(
        body,
        grid=(grid,),
        in_specs=[
            pl.BlockSpec((br, d), lambda i: (i, 0)),
            pl.BlockSpec((d, d), lambda i: (0, 0)),
            pl.BlockSpec((d, d), lambda i: (0, 0)),
            pl.BlockSpec((1, d), lambda i: (0, 0)),
        ],
        out_specs=[
            pl.BlockSpec((br, d), lambda i: (i, 0)),
            pl.BlockSpec((br, d), lambda i: (i, 0)),
        ],
        out_shape=[
            jax.ShapeDtypeStruct((npad, d), jnp.float32),
            jax.ShapeDtypeStruct((npad, d), jnp.float32),
        ],
    )(xp, w_l, w_r, b.reshape(1, d))


def _combine_transform(p, c, r1, w_l, w_r, b, br):
    npad, d = r1.shape
    grid = npad // br

    def body(p_ref, c_ref, r1_ref, wl_ref, wr_ref, b_ref, z_ref, r_ref):
        agg = p_ref[0] + p_ref[1]
        cnt = jnp.maximum(c_ref[0] + c_ref[1], 1.0).reshape(br, 1)
        h = jnp.maximum(agg / cnt + r1_ref[...], 0.0)
        z_ref[...] = _dot_t(h, wl_ref[...])
        r_ref[...] = _dot_t(h, wr_ref[...]) + b_ref[...]

    return pl.pallas_call(
        body,
        grid=(grid,),
        in_specs=[
            pl.BlockSpec((2, br, d), lambda i: (0, i, 0)),
            pl.BlockSpec((2, br), lambda i: (0, i)),
            pl.BlockSpec((br, d), lambda i: (i, 0)),
            pl.BlockSpec((d, d), lambda i: (0, 0)),
            pl.BlockSpec((d, d), lambda i: (0, 0)),
            pl.BlockSpec((1, d), lambda i: (0, 0)),
        ],
        out_specs=[
            pl.BlockSpec((br, d), lambda i: (i, 0)),
            pl.BlockSpec((br, d), lambda i: (i, 0)),
        ],
        out_shape=[
            jax.ShapeDtypeStruct((npad, d), jnp.float32),
            jax.ShapeDtypeStruct((npad, d), jnp.float32),
        ],
    )(p, c, r1, w_l, w_r, b.reshape(1, d))


def _final(p, c, r2, br):
    npad, d = r2.shape
    grid = npad // br

    def body(p_ref, c_ref, r2_ref, o_ref):
        agg = p_ref[0] + p_ref[1]
        cnt = jnp.maximum(c_ref[0] + c_ref[1], 1.0).reshape(br, 1)
        o_ref[...] = agg / cnt + r2_ref[...]

    return pl.pallas_call(
        body,
        grid=(grid,),
        in_specs=[
            pl.BlockSpec((2, br, d), lambda i: (0, i, 0)),
            pl.BlockSpec((2, br), lambda i: (0, i)),
            pl.BlockSpec((br, d), lambda i: (i, 0)),
        ],
        out_specs=pl.BlockSpec((br, d), lambda i: (i, 0)),
        out_shape=jax.ShapeDtypeStruct((npad, d), jnp.float32),
    )(p, c, r2)


def _sc_scatter(z, src, dst, zeros_c, nch, compute_counts):
    npad, d = z.shape
    rows_pt = npad // NS
    mesh = plsc.VectorSubcoreMesh(core_axis_name="c", subcore_axis_name="s")

    out_type = [jax.ShapeDtypeStruct((NC, npad, d), jnp.float32)]
    if compute_counts:
        out_type.append(jax.ShapeDtypeStruct((NC * npad,), jnp.float32))

    scratch = dict(
        agg_sh=pltpu.VMEM_SHARED((npad, d), jnp.float32),
        src_v=pltpu.VMEM((CH,), jnp.int32),
        dst_v=pltpu.VMEM((CH,), jnp.int32),
        rows_v=pltpu.VMEM((CH, d), jnp.float32),
        sem=pltpu.SemaphoreType.DMA,
    )
    if compute_counts:
        scratch.update(
            cnt_sh=pltpu.VMEM_SHARED((npad,), jnp.float32),
            ones_v=pltpu.VMEM((CH,), jnp.float32),
            zc_v=pltpu.VMEM((rows_pt,), jnp.float32),
        )

    @functools.partial(pl.kernel, out_type=out_type, mesh=mesh,
                       scratch_types=scratch)
    def k(z_hbm, src_hbm, dst_hbm, zeros_hbm, p_hbm, *rest,
          agg_sh, src_v, dst_v, rows_v, sem,
          cnt_sh=None, ones_v=None, zc_v=None):
        cid = lax.axis_index("c")
        sid = lax.axis_index("s")
        wid = sid * NC + cid
        r0 = sid * rows_pt

        pltpu.sync_copy(zeros_hbm, agg_sh.at[pl.ds(r0, rows_pt)])
        if compute_counts:
            one16 = jnp.ones((16,), jnp.float32)
            zer16 = jnp.zeros((16,), jnp.float32)

            @pl.loop(0, CH // 16)
            def _(i):
                ones_v[pl.ds(i * 16, 16)] = one16

            @pl.loop(0, rows_pt // 16)
            def _(i):
                zc_v[pl.ds(i * 16, 16)] = zer16

            pltpu.sync_copy(zc_v, cnt_sh.at[pl.ds(r0, rows_pt)])
        plsc.subcore_barrier()

        @pl.loop(0, nch)
        def _(ch):
            base = (wid * nch + ch) * CH
            pltpu.sync_copy(src_hbm.at[pl.ds(base, CH)], src_v)
            pltpu.sync_copy(dst_hbm.at[pl.ds(base, CH)], dst_v)
            pltpu.async_copy(z_hbm.at[src_v], rows_v, sem).wait()
            pltpu.sync_copy(rows_v, agg_sh.at[dst_v], add=True)
            if compute_counts:
                pltpu.sync_copy(ones_v, cnt_sh.at[dst_v], add=True)

        plsc.subcore_barrier()
        pltpu.sync_copy(agg_sh.at[pl.ds(r0, rows_pt)],
                        p_hbm.at[cid, pl.ds(r0, rows_pt)])
        if compute_counts:
            (c_hbm,) = rest
            pltpu.sync_copy(cnt_sh.at[pl.ds(r0, rows_pt)], zc_v)
            pltpu.sync_copy(zc_v, c_hbm.at[pl.ds(cid * npad + r0, rows_pt)])

    return k(z, src, dst, zeros_c)


def kernel(x, edge_index, W1_l, W1_r, b1, W2_l, W2_r, b2):
    n, d = x.shape
    e = edge_index.shape[1]

    npad = ((n + 1 + 255) // 256) * 256
    rows_pt = npad // NS
    br = npad // 8

    per_w = ((e + NW * CH - 1) // (NW * CH)) * CH
    nch = per_w // CH
    e_pad = per_w * NW

    src = jnp.pad(edge_index[0].astype(jnp.int32), (0, e_pad - e))
    dst = jnp.pad(edge_index[1].astype(jnp.int32), (0, e_pad - e),
                  constant_values=n)
    xp = jnp.pad(x, ((0, npad - n), (0, 0)))

    zeros_c = jnp.zeros((rows_pt, d), jnp.float32)

    z1, r1 = _transform(xp, W1_l, W1_r, b1, br)
    p1, c = _sc_scatter(z1, src, dst, zeros_c, nch, True)
    c2 = c.reshape(NC, npad)
    z2, r2 = _combine_transform(p1, c2, r1, W2_l, W2_r, b2, br)
    (p2,) = _sc_scatter(z2, src, dst, zeros_c, nch, False)
    out = _final(p2, c2, r2, br)
    return out[:n]
